# baseline (speedup 1.0000x reference)
import functools

import jax
import jax.numpy as jnp
from jax.experimental import pallas as pl

_ANCHORS = ((0.28, 0.22), (0.38, 0.48), (0.9, 0.78))
_NA = 3


def _yolo_kernel(x_ref, boxes_ref, conf_ref, cls_ref, *, H, W, aw, ah):
    P = H * W
    a = pl.program_id(1)
    s = x_ref[0, 0]
    conf_ref[0, 0] = jax.nn.sigmoid(s[4:5, :])
    t = s.T
    u = t[:, 0:4]
    col = jax.lax.broadcasted_iota(jnp.int32, (P, 4), 1)
    row = jax.lax.broadcasted_iota(jnp.int32, (P, 4), 0)
    gx = (row // W).astype(jnp.float32)
    gy = (row % W).astype(jnp.float32)
    off = jnp.where(col == 0, gx, jnp.where(col == 1, gy, 0.0))
    aw_s = jnp.where(a == 0, aw[0], jnp.where(a == 1, aw[1], aw[2]))
    ah_s = jnp.where(a == 0, ah[0], jnp.where(a == 1, ah[1], ah[2]))
    anch = jnp.where(col == 2, aw_s, ah_s)
    inv = jnp.where(col % 2 == 0, 1.0 / H, 1.0 / W).astype(jnp.float32)
    boxes = jnp.where(col < 2, jax.nn.sigmoid(u) + off, jnp.exp(u) * anch)
    boxes_ref[0, 0] = boxes * inv
    cls_ref[0, 0] = jax.nn.sigmoid(t[:, 5:])


def kernel(x):
    B, C, H, W = x.shape
    nA = _NA
    nCp5 = C // nA
    nC = nCp5 - 5
    P = H * W
    xr = x.reshape(B, nA, nCp5, P)
    aw = tuple(float(a0) * H for (a0, _) in _ANCHORS)
    ah = tuple(float(a1) * W for (_, a1) in _ANCHORS)
    out_shapes = (
        jax.ShapeDtypeStruct((B, nA, P, 4), jnp.float32),
        jax.ShapeDtypeStruct((B, nA, 1, P), jnp.float32),
        jax.ShapeDtypeStruct((B, nA, P, nC), jnp.float32),
    )
    boxes, conf, cls_ = pl.pallas_call(
        functools.partial(_yolo_kernel, H=H, W=W, aw=aw, ah=ah),
        grid=(B, nA),
        in_specs=[pl.BlockSpec((1, 1, nCp5, P), lambda b, a: (b, a, 0, 0))],
        out_specs=(
            pl.BlockSpec((1, 1, P, 4), lambda b, a: (b, a, 0, 0)),
            pl.BlockSpec((1, 1, 1, P), lambda b, a: (b, a, 0, 0)),
            pl.BlockSpec((1, 1, P, nC), lambda b, a: (b, a, 0, 0)),
        ),
        out_shape=out_shapes,
    )(xr)
    return (boxes.reshape(B, nA, H, W, 4),
            conf.reshape(B, nA, H, W),
            cls_.reshape(B, nA, H, W, nC))


# speedup vs baseline: 1.1997x; 1.1997x over previous
import functools

import jax
import jax.numpy as jnp
from jax.experimental import pallas as pl

_ANCHORS = ((0.28, 0.22), (0.38, 0.48), (0.9, 0.78))
_NA = 3
_BPB = 4


def _yolo_kernel(x_ref, boxes_ref, conf_ref, cls_ref, *, H, W, aw, ah):
    s = x_ref[...]
    hd = s[:, :, 0:4, :]
    shp = hd.shape
    aid = jax.lax.broadcasted_iota(jnp.int32, shp, 1)
    rid = jax.lax.broadcasted_iota(jnp.int32, shp, 2)
    lan = jax.lax.broadcasted_iota(jnp.int32, shp, 3)
    gx = (lan // W).astype(jnp.float32)
    gy = (lan % W).astype(jnp.float32)
    off = jnp.where(rid == 0, gx, jnp.where(rid == 1, gy, 0.0))
    aw_v = jnp.where(aid == 0, aw[0], jnp.where(aid == 1, aw[1], aw[2]))
    ah_v = jnp.where(aid == 0, ah[0], jnp.where(aid == 1, ah[1], ah[2]))
    anch = jnp.where(rid == 2, aw_v, ah_v)
    inv = jnp.where(rid % 2 == 0, 1.0 / H, 1.0 / W).astype(jnp.float32)
    dec = jnp.where(rid < 2, jax.nn.sigmoid(hd) + off, jnp.exp(hd) * anch)
    boxes_ref[...] = jnp.transpose(dec * inv, (0, 1, 3, 2))
    conf_ref[...] = jax.nn.sigmoid(s[:, :, 4:5, :])
    cls_ref[...] = jnp.transpose(jax.nn.sigmoid(s[:, :, 5:, :]), (0, 1, 3, 2))


def kernel(x):
    B, C, H, W = x.shape
    nA = _NA
    nCp5 = C // nA
    nC = nCp5 - 5
    P = H * W
    bpb = _BPB
    xr = x.reshape(B, nA, nCp5, P)
    aw = tuple(float(a0) * H for (a0, _) in _ANCHORS)
    ah = tuple(float(a1) * W for (_, a1) in _ANCHORS)
    out_shapes = (
        jax.ShapeDtypeStruct((B, nA, P, 4), jnp.float32),
        jax.ShapeDtypeStruct((B, nA, 1, P), jnp.float32),
        jax.ShapeDtypeStruct((B, nA, P, nC), jnp.float32),
    )
    boxes, conf, cls_ = pl.pallas_call(
        functools.partial(_yolo_kernel, H=H, W=W, aw=aw, ah=ah),
        grid=(B // bpb,),
        in_specs=[pl.BlockSpec((bpb, nA, nCp5, P), lambda b: (b, 0, 0, 0))],
        out_specs=(
            pl.BlockSpec((bpb, nA, P, 4), lambda b: (b, 0, 0, 0)),
            pl.BlockSpec((bpb, nA, 1, P), lambda b: (b, 0, 0, 0)),
            pl.BlockSpec((bpb, nA, P, nC), lambda b: (b, 0, 0, 0)),
        ),
        out_shape=out_shapes,
    )(xr)
    return (boxes.reshape(B, nA, H, W, 4),
            conf.reshape(B, nA, H, W),
            cls_.reshape(B, nA, H, W, nC))
